# baseline (speedup 1.0000x reference)
import functools

import jax
import jax.numpy as jnp
from jax import lax
from jax.experimental import pallas as pl
from jax.experimental.pallas import tpu as pltpu
from jax.experimental.pallas import tpu_sc as plsc

_TABLE = 120000
_B = 16384
_H = 200
_NW = 32
_C = 1024
_CPR = _B // _C
_NCHUNK = _H * _CPR // _NW
_NB = 4
_NGRP = _NCHUNK // _NB

_mesh = plsc.VectorSubcoreMesh(core_axis_name="c", subcore_axis_name="s")


@functools.partial(
    pl.kernel,
    mesh=_mesh,
    out_type=jax.ShapeDtypeStruct((_H, _B), jnp.int32),
    compiler_params=pltpu.CompilerParams(needs_layout_passes=False),
    scratch_types=[
        pltpu.VMEM_SHARED((_TABLE,), jnp.int32),
        [pltpu.VMEM((1, _C), jnp.int32)] * _NB,
        pltpu.VMEM((1, _C), jnp.int32),
        [pltpu.SemaphoreType.DMA] * _NB,
        pltpu.SemaphoreType.DMA,
        pltpu.SemaphoreType.DMA,
    ],
)
def _sc_gather(idx_hbm, table_hbm, out_hbm, table_s, bufs, gbuf,
               sem_in, sem_out, gsem):
    wid = lax.axis_index("s") * 2 + lax.axis_index("c")
    k0 = wid * _NCHUNK

    def in_copy(c, j):
        k = k0 + c
        return pltpu.make_async_copy(
            idx_hbm.at[pl.ds(k // _CPR, 1), pl.ds((k % _CPR) * _C, _C)],
            bufs[j], sem_in[j])

    def out_copy(c):
        k = k0 + c
        return pltpu.make_async_copy(
            gbuf,
            out_hbm.at[pl.ds(k // _CPR, 1), pl.ds((k % _CPR) * _C, _C)],
            sem_out)

    _DEPTH = _NB - 1

    for c in range(_DEPTH):
        in_copy(c, c).start()

    sid = lax.axis_index("s")

    @pl.when(sid == 0)
    def _():
        pltpu.sync_copy(table_hbm, table_s)

    plsc.subcore_barrier()

    def group(g, carry):
        for j in range(_NB):
            c = g * _NB + j
            jp = (j + _DEPTH) % _NB

            @pl.when(c + _DEPTH < _NCHUNK)
            def _():
                in_copy(c + _DEPTH, jp).start()

            in_copy(c, j).wait()

            pltpu.async_copy(
                table_s.at[bufs[j].at[0]], gbuf.at[0], gsem).wait()

            out_copy(c).start()
            out_copy(c).wait()
        return carry

    lax.fori_loop(0, _NGRP, group, 0)


def kernel(inputs, table):
    idx_t = jnp.transpose(inputs.astype(jnp.int32))
    return jnp.transpose(_sc_gather(idx_t, table))


# speedup vs baseline: 1.5135x; 1.5135x over previous
import functools

import jax
import jax.numpy as jnp
from jax import lax
from jax.experimental import pallas as pl
from jax.experimental.pallas import tpu as pltpu
from jax.experimental.pallas import tpu_sc as plsc

_TABLE = 120000
_B = 16384
_H = 200
_NW = 32
_C = 2048
_CPR = _B // _C
_NCHUNK = _H * _CPR // _NW
_NB = 5
_NGRP = _NCHUNK // _NB

_mesh = plsc.VectorSubcoreMesh(core_axis_name="c", subcore_axis_name="s")


@functools.partial(
    pl.kernel,
    mesh=_mesh,
    out_type=jax.ShapeDtypeStruct((_H, _B), jnp.int32),
    compiler_params=pltpu.CompilerParams(needs_layout_passes=False),
    scratch_types=[
        pltpu.VMEM((_TABLE,), jnp.int32),
        [pltpu.VMEM((1, _C), jnp.int32)] * _NB,
        [pltpu.SemaphoreType.DMA] * _NB,
        [pltpu.SemaphoreType.DMA] * _NB,
    ],
)
def _sc_gather(idx_hbm, table_hbm, out_hbm, table_v, bufs, sem_in, sem_out):
    wid = lax.axis_index("s") * 2 + lax.axis_index("c")
    k0 = wid * _NCHUNK

    def in_copy(c, j):
        k = k0 + c
        return pltpu.make_async_copy(
            idx_hbm.at[pl.ds(k // _CPR, 1), pl.ds((k % _CPR) * _C, _C)],
            bufs[j], sem_in[j])

    def out_copy(c, j):
        k = k0 + c
        return pltpu.make_async_copy(
            bufs[j],
            out_hbm.at[pl.ds(k // _CPR, 1), pl.ds((k % _CPR) * _C, _C)],
            sem_out[j])

    _DEPTH = _NB - 2

    for c in range(_DEPTH):
        in_copy(c, c).start()
    pltpu.sync_copy(table_hbm, table_v)

    def group(g, carry):
        for j in range(_NB):
            c = g * _NB + j
            jp = (j + _DEPTH) % _NB

            @pl.when(c >= 2)
            def _():
                out_copy(c - 2, jp).wait()

            @pl.when(c + _DEPTH < _NCHUNK)
            def _():
                in_copy(c + _DEPTH, jp).start()

            in_copy(c, j).wait()

            @plsc.parallel_loop(0, _C // 16, unroll=8)
            def vec_body(i):
                v = bufs[j][0, pl.ds(i * 16, 16)]
                bufs[j][0, pl.ds(i * 16, 16)] = plsc.load_gather(
                    table_v, [v])

            out_copy(c, j).start()
        return carry

    lax.fori_loop(0, _NGRP, group, 0)

    for c in range(_NCHUNK - 2, _NCHUNK):
        out_copy(c, c % _NB).wait()


def kernel(inputs, table):
    idx_t = jnp.transpose(inputs.astype(jnp.int32))
    return jnp.transpose(_sc_gather(idx_t, table))


# speedup vs baseline: 1.5469x; 1.0221x over previous
import functools

import jax
import jax.numpy as jnp
from jax import lax
from jax.experimental import pallas as pl
from jax.experimental.pallas import tpu as pltpu
from jax.experimental.pallas import tpu_sc as plsc

_TABLE = 120000
_B = 16384
_H = 200
_NW = 32
_C = 1024
_CPR = _B // _C
_NCHUNK = _H * _CPR // _NW
_NB = 10
_NGRP = _NCHUNK // _NB

_mesh = plsc.VectorSubcoreMesh(core_axis_name="c", subcore_axis_name="s")


@functools.partial(
    pl.kernel,
    mesh=_mesh,
    out_type=jax.ShapeDtypeStruct((_H, _B), jnp.int32),
    compiler_params=pltpu.CompilerParams(needs_layout_passes=False),
    scratch_types=[
        pltpu.VMEM((_TABLE,), jnp.int32),
        [pltpu.VMEM((1, _C), jnp.int32)] * _NB,
        [pltpu.SemaphoreType.DMA] * _NB,
        [pltpu.SemaphoreType.DMA] * _NB,
    ],
)
def _sc_gather(idx_hbm, table_hbm, out_hbm, table_v, bufs, sem_in, sem_out):
    wid = lax.axis_index("s") * 2 + lax.axis_index("c")
    k0 = wid * _NCHUNK

    def in_copy(c, j):
        k = k0 + c
        return pltpu.make_async_copy(
            idx_hbm.at[pl.ds(k // _CPR, 1), pl.ds((k % _CPR) * _C, _C)],
            bufs[j], sem_in[j])

    def out_copy(c, j):
        k = k0 + c
        return pltpu.make_async_copy(
            bufs[j],
            out_hbm.at[pl.ds(k // _CPR, 1), pl.ds((k % _CPR) * _C, _C)],
            sem_out[j])

    _DEPTH = _NB - 2

    for c in range(_DEPTH):
        in_copy(c, c).start()
    pltpu.sync_copy(table_hbm, table_v)

    def group(g, carry):
        for j in range(_NB):
            c = g * _NB + j
            jp = (j + _DEPTH) % _NB

            @pl.when(c >= 2)
            def _():
                out_copy(c - 2, jp).wait()

            @pl.when(c + _DEPTH < _NCHUNK)
            def _():
                in_copy(c + _DEPTH, jp).start()

            in_copy(c, j).wait()

            @plsc.parallel_loop(0, _C // 16, unroll=8)
            def vec_body(i):
                v = bufs[j][0, pl.ds(i * 16, 16)]
                bufs[j][0, pl.ds(i * 16, 16)] = plsc.load_gather(
                    table_v, [v])

            out_copy(c, j).start()
        return carry

    lax.fori_loop(0, _NGRP, group, 0)

    for c in range(_NCHUNK - 2, _NCHUNK):
        out_copy(c, c % _NB).wait()


def kernel(inputs, table):
    idx_t = jnp.transpose(inputs.astype(jnp.int32))
    return jnp.transpose(_sc_gather(idx_t, table))
